# Initial kernel scaffold; baseline (speedup 1.0000x reference)
import jax
import jax.numpy as jnp
from jax.experimental import pallas as pl


def kernel(x, rel_pos_emb_weight):
    raise NotImplementedError("write your pallas kernel here")


# speedup vs baseline: 7.3545x; 7.3545x over previous
import functools

import jax
import jax.numpy as jnp
from jax.experimental import pallas as pl
from jax.experimental.pallas import tpu as pltpu

_MAX_REL = 32


def _body(x_ref, w_ref, o_ref, *, bt, seq, rmax):
    i = pl.program_id(0)
    nk = 2 * rmax + 1
    b = jax.lax.broadcasted_iota(jnp.int32, (bt, nk), 0) + i * bt
    k = jax.lax.broadcasted_iota(jnp.int32, (bt, nk), 1)
    j = b + (k - rmax)
    interior = ((k > 0) & (k < 2 * rmax) & (j >= 0) & (j < seq)).astype(jnp.int32)
    counts = jnp.where(
        k == 0,
        jnp.maximum(b - (rmax - 1), 0),
        jnp.where(k == 2 * rmax, jnp.maximum(seq - rmax - b, 0), interior),
    ).astype(jnp.float32)
    bias = jnp.dot(counts, w_ref[...], preferred_element_type=jnp.float32)
    bias = bias * (1.0 / seq)
    o_ref[...] = x_ref[...] + bias[:, None, :]


def kernel(x, rel_pos_emb_weight):
    batch, seq, d = x.shape
    bt = 16
    grid = (batch // bt,)
    return pl.pallas_call(
        functools.partial(_body, bt=bt, seq=seq, rmax=_MAX_REL),
        grid=grid,
        in_specs=[
            pl.BlockSpec((bt, seq, d), lambda i: (i, 0, 0)),
            pl.BlockSpec(rel_pos_emb_weight.shape, lambda i: (0, 0)),
        ],
        out_specs=pl.BlockSpec((bt, seq, d), lambda i: (i, 0, 0)),
        out_shape=jax.ShapeDtypeStruct((batch, seq, d), x.dtype),
        compiler_params=pltpu.CompilerParams(
            dimension_semantics=("arbitrary",),
        ),
    )(x, rel_pos_emb_weight)
